# baseline (speedup 1.0000x reference)
import functools

import jax
import jax.numpy as jnp
from jax import lax
from jax.experimental import pallas as pl
from jax.experimental.pallas import tpu as pltpu
from jax.experimental.pallas import tpu_sc as plsc

_C = 1600
_G = 128


def _sc_gather(table, idx_flat, n_rows, d):
    info = plsc.get_sparse_core_info()
    nw = info.num_cores * info.num_subcores
    per_w = n_rows // nw
    n_steps = per_w // _C
    assert per_w % _C == 0
    n_full = _C // _G
    rem = _C % _G

    mesh = plsc.VectorSubcoreMesh(core_axis_name="c", subcore_axis_name="s")

    @functools.partial(
        pl.kernel,
        mesh=mesh,
        compiler_params=pltpu.CompilerParams(use_tc_tiling_on_sc=False),
        out_type=jax.ShapeDtypeStruct((n_rows // 2, 2 * d), jnp.float32),
        scratch_types=[
            pltpu.VMEM((_C,), jnp.int32),
            pltpu.VMEM((_C, d), jnp.float32),
            pltpu.SemaphoreType.DMA,
        ],
    )
    def gather_kernel(idx_hbm, table_hbm, out_hbm, idx_v, rows_v, sem):
        wid = lax.axis_index("s") * info.num_cores + lax.axis_index("c")

        def step_body(step, carry):
            tok0 = pl.multiple_of(wid * per_w + step * _C, _C)
            pltpu.sync_copy(idx_hbm.at[pl.ds(tok0, _C)], idx_v)
            copies = []
            for g in range(n_full):
                copies.append(
                    pltpu.async_copy(
                        table_hbm.at[idx_v.at[pl.ds(g * _G, _G)]],
                        rows_v.at[pl.ds(g * _G, _G)],
                        sem,
                    )
                )
            if rem:
                copies.append(
                    pltpu.async_copy(
                        table_hbm.at[idx_v.at[pl.ds(n_full * _G, rem)]],
                        rows_v.at[pl.ds(n_full * _G, rem)],
                        sem,
                    )
                )
            for c in copies:
                c.wait()
            blk = wid * n_steps + step
            row0 = pl.multiple_of((blk // 4) * 2 * _C + (blk % 2) * _C, _C)
            col0 = pl.multiple_of(((blk % 4) // 2) * d, d)
            pltpu.sync_copy(rows_v, out_hbm.at[pl.ds(row0, _C), pl.ds(col0, d)])
            return carry

        lax.fori_loop(0, n_steps, step_body, 0)

    return gather_kernel(idx_flat, table)


def _make_mlp_body(BB, L, d, aliased):
    def _mlp_body(*refs):
        x_ref, w1_ref, b1_ref, w2_ref, b2_ref, w3_ref, b3_ref = refs[:7]
        o_ref = refs[-1]
        x2 = x_ref[...]
        x = jnp.concatenate([x2[:, :d], x2[:, d:]], axis=0)
        h = jnp.dot(x, w1_ref[...], preferred_element_type=jnp.float32) + b1_ref[...]
        h = jnp.maximum(h, 0.0)
        h = jnp.dot(h, w2_ref[...], preferred_element_type=jnp.float32) + b2_ref[...]
        h = jnp.tanh(h)
        o = jnp.dot(h, w3_ref[...], preferred_element_type=jnp.float32) + b3_ref[...]
        o_t = o.T
        for l in range(L):
            o_ref[l] = o_t[:, l * BB:(l + 1) * BB]
    return _mlp_body


def _tc_mlp_chunk(embs2, w1t, b1, w2t, b2, w3t, b3, B, L, chunk, n_chunks,
                  prev_out):
    d_in = embs2.shape[1] // 2
    h1 = w1t.shape[1]
    h2 = w2t.shape[1]
    d_out = w3t.shape[1]
    BB = 4 * _C // L
    blocks = B // BB // n_chunks
    base = chunk * blocks
    in_specs = [
        pl.BlockSpec((2 * _C, 2 * d_in), lambda i: (i, 0)),
        pl.BlockSpec((d_in, h1), lambda i: (0, 0)),
        pl.BlockSpec((1, h1), lambda i: (0, 0)),
        pl.BlockSpec((h1, h2), lambda i: (0, 0)),
        pl.BlockSpec((1, h2), lambda i: (0, 0)),
        pl.BlockSpec((h2, d_out), lambda i: (0, 0)),
        pl.BlockSpec((1, d_out), lambda i: (0, 0)),
    ]
    args = [embs2, w1t, b1.reshape(1, -1), w2t, b2.reshape(1, -1), w3t,
            b3.reshape(1, -1)]
    aliases = {}
    if prev_out is not None:
        in_specs.append(pl.BlockSpec(memory_space=pl.ANY))
        args.append(prev_out)
        aliases = {7: 0}
    return pl.pallas_call(
        _make_mlp_body(BB, L, d_in, prev_out is not None),
        grid=(blocks,),
        in_specs=in_specs,
        out_specs=pl.BlockSpec((L, d_out, BB), lambda i: (0, 0, base + i)),
        out_shape=jax.ShapeDtypeStruct((L, d_out, B), jnp.float32),
        input_output_aliases=aliases,
        compiler_params=pltpu.CompilerParams(
            dimension_semantics=("arbitrary",),
        ),
    )(*args)


def kernel(indices, table, W1, b1, W2, b2, W3, b3):
    B, L = indices.shape
    d = table.shape[1]
    n_rows = B * L
    BB = 4 * _C // L
    n_chunks = 2
    rows_c = n_rows // n_chunks
    idx_perm = (
        indices.reshape(B // BB, BB, L)
        .transpose(0, 2, 1)
        .reshape(n_rows)
        .astype(jnp.int32)
    )
    w1t, w2t, w3t = W1.T, W2.T, W3.T
    table = table + (b1[0] - b1[0])
    stagings = [
        _sc_gather(table, lax.slice(idx_perm, (c * rows_c,), ((c + 1) * rows_c,)),
                   rows_c, d)
        for c in range(n_chunks)
    ]
    out_t = None
    for c in range(n_chunks):
        out_t = _tc_mlp_chunk(stagings[c], w1t, b1, w2t, b2, w3t, b3, B, L,
                              c, n_chunks, out_t)
    return jnp.transpose(out_t, (2, 0, 1))


# speedup vs baseline: 1.3531x; 1.3531x over previous
import functools

import jax
import jax.numpy as jnp
from jax import lax
from jax.experimental import pallas as pl
from jax.experimental.pallas import tpu as pltpu
from jax.experimental.pallas import tpu_sc as plsc

_C = 1600
_G = 128


def _sc_gather(table, idx_flat, n_rows, d):
    info = plsc.get_sparse_core_info()
    nw = info.num_cores * info.num_subcores
    per_w = n_rows // nw
    n_steps = per_w // _C
    assert per_w % _C == 0
    n_full = _C // _G
    rem = _C % _G

    mesh = plsc.VectorSubcoreMesh(core_axis_name="c", subcore_axis_name="s")

    @functools.partial(
        pl.kernel,
        mesh=mesh,
        compiler_params=pltpu.CompilerParams(use_tc_tiling_on_sc=False),
        out_type=jax.ShapeDtypeStruct((n_rows // 2, 2 * d), jnp.float32),
        scratch_types=[
            pltpu.VMEM((_C,), jnp.int32),
            pltpu.VMEM((_C, d), jnp.float32),
            pltpu.SemaphoreType.DMA,
        ],
    )
    def gather_kernel(idx_hbm, table_hbm, out_hbm, idx_v, rows_v, sem):
        wid = lax.axis_index("s") * info.num_cores + lax.axis_index("c")

        def step_body(step, carry):
            tok0 = pl.multiple_of(wid * per_w + step * _C, _C)
            pltpu.sync_copy(idx_hbm.at[pl.ds(tok0, _C)], idx_v)
            copies = []
            for g in range(n_full):
                copies.append(
                    pltpu.async_copy(
                        table_hbm.at[idx_v.at[pl.ds(g * _G, _G)]],
                        rows_v.at[pl.ds(g * _G, _G)],
                        sem,
                    )
                )
            if rem:
                copies.append(
                    pltpu.async_copy(
                        table_hbm.at[idx_v.at[pl.ds(n_full * _G, rem)]],
                        rows_v.at[pl.ds(n_full * _G, rem)],
                        sem,
                    )
                )
            for c in copies:
                c.wait()
            blk = wid * n_steps + step
            row0 = pl.multiple_of((blk // 4) * 2 * _C + (blk % 2) * _C, _C)
            col0 = pl.multiple_of(((blk % 4) // 2) * d, d)
            pltpu.sync_copy(rows_v, out_hbm.at[pl.ds(row0, _C), pl.ds(col0, d)])
            return carry

        lax.fori_loop(0, n_steps, step_body, 0)

    return gather_kernel(idx_flat, table)


def _make_mlp_body(BB, L, d, aliased):
    def _mlp_body(*refs):
        x_ref, w1_ref, b1_ref, w2_ref, b2_ref, w3_ref, b3_ref = refs[:7]
        o_ref = refs[-1]
        x2 = x_ref[...]
        x = jnp.concatenate([x2[:, :d], x2[:, d:]], axis=0)
        h = jnp.dot(x, w1_ref[...], preferred_element_type=jnp.float32) + b1_ref[...]
        h = jnp.maximum(h, 0.0)
        h = jnp.dot(h, w2_ref[...], preferred_element_type=jnp.float32) + b2_ref[...]
        h = jnp.tanh(h)
        o = jnp.dot(h, w3_ref[...], preferred_element_type=jnp.float32) + b3_ref[...]
        o_t = o.T
        for l in range(L):
            o_ref[l] = o_t[:, l * BB:(l + 1) * BB]
    return _mlp_body


def _tc_mlp_chunk(embs2, w1t, b1, w2t, b2, w3t, b3, B, L, chunk, n_chunks,
                  prev_out):
    d_in = embs2.shape[1] // 2
    h1 = w1t.shape[1]
    h2 = w2t.shape[1]
    d_out = w3t.shape[1]
    BB = 4 * _C // L
    blocks = B // BB // n_chunks
    base = chunk * blocks
    in_specs = [
        pl.BlockSpec((2 * _C, 2 * d_in), lambda i: (i, 0)),
        pl.BlockSpec((d_in, h1), lambda i: (0, 0)),
        pl.BlockSpec((1, h1), lambda i: (0, 0)),
        pl.BlockSpec((h1, h2), lambda i: (0, 0)),
        pl.BlockSpec((1, h2), lambda i: (0, 0)),
        pl.BlockSpec((h2, d_out), lambda i: (0, 0)),
        pl.BlockSpec((1, d_out), lambda i: (0, 0)),
    ]
    args = [embs2, w1t, b1.reshape(1, -1), w2t, b2.reshape(1, -1), w3t,
            b3.reshape(1, -1)]
    aliases = {}
    if prev_out is not None:
        in_specs.append(pl.BlockSpec(memory_space=pl.ANY))
        args.append(prev_out)
        aliases = {7: 0}
    return pl.pallas_call(
        _make_mlp_body(BB, L, d_in, prev_out is not None),
        grid=(blocks,),
        in_specs=in_specs,
        out_specs=pl.BlockSpec((L, d_out, BB), lambda i: (0, 0, base + i)),
        out_shape=jax.ShapeDtypeStruct((L, d_out, B), jnp.float32),
        input_output_aliases=aliases,
        compiler_params=pltpu.CompilerParams(
            dimension_semantics=("arbitrary",),
        ),
    )(*args)


def kernel(indices, table, W1, b1, W2, b2, W3, b3):
    B, L = indices.shape
    d = table.shape[1]
    n_rows = B * L
    BB = 4 * _C // L
    n_chunks = 4
    rows_c = n_rows // n_chunks
    idx_perm = (
        indices.reshape(B // BB, BB, L)
        .transpose(0, 2, 1)
        .reshape(n_rows)
        .astype(jnp.int32)
    )
    w1t, w2t, w3t = W1.T, W2.T, W3.T
    stagings = [
        _sc_gather(table, lax.slice(idx_perm, (c * rows_c,), ((c + 1) * rows_c,)),
                   rows_c, d)
        for c in range(n_chunks)
    ]
    out_t = None
    for c in range(n_chunks):
        out_t = _tc_mlp_chunk(stagings[c], w1t, b1, w2t, b2, w3t, b3, B, L,
                              c, n_chunks, out_t)
    return jnp.transpose(out_t, (2, 0, 1))


# speedup vs baseline: 1.3656x; 1.0093x over previous
import functools

import jax
import jax.numpy as jnp
from jax import lax
from jax.experimental import pallas as pl
from jax.experimental.pallas import tpu as pltpu
from jax.experimental.pallas import tpu_sc as plsc

_C = 1600
_G = 128


def _sc_gather(table, idx_flat, n_rows, d):
    info = plsc.get_sparse_core_info()
    nw = info.num_cores * info.num_subcores
    per_w = n_rows // nw
    n_steps = per_w // _C
    assert per_w % _C == 0
    n_full = _C // _G
    rem = _C % _G

    mesh = plsc.VectorSubcoreMesh(core_axis_name="c", subcore_axis_name="s")

    @functools.partial(
        pl.kernel,
        mesh=mesh,
        compiler_params=pltpu.CompilerParams(use_tc_tiling_on_sc=False),
        out_type=jax.ShapeDtypeStruct((n_rows // 2, 2 * d), jnp.float32),
        scratch_types=[
            pltpu.VMEM((_C,), jnp.int32),
            pltpu.VMEM((_C, d), jnp.float32),
            pltpu.SemaphoreType.DMA,
        ],
    )
    def gather_kernel(idx_hbm, table_hbm, out_hbm, idx_v, rows_v, sem):
        wid = lax.axis_index("s") * info.num_cores + lax.axis_index("c")

        def step_body(step, carry):
            tok0 = pl.multiple_of(wid * per_w + step * _C, _C)
            pltpu.sync_copy(idx_hbm.at[pl.ds(tok0, _C)], idx_v)
            copies = []
            for g in range(n_full):
                copies.append(
                    pltpu.async_copy(
                        table_hbm.at[idx_v.at[pl.ds(g * _G, _G)]],
                        rows_v.at[pl.ds(g * _G, _G)],
                        sem,
                    )
                )
            if rem:
                copies.append(
                    pltpu.async_copy(
                        table_hbm.at[idx_v.at[pl.ds(n_full * _G, rem)]],
                        rows_v.at[pl.ds(n_full * _G, rem)],
                        sem,
                    )
                )
            for c in copies:
                c.wait()
            blk = wid * n_steps + step
            row0 = pl.multiple_of((blk // 4) * 2 * _C + (blk % 2) * _C, _C)
            col0 = pl.multiple_of(((blk % 4) // 2) * d, d)
            pltpu.sync_copy(rows_v, out_hbm.at[pl.ds(row0, _C), pl.ds(col0, d)])
            return carry

        lax.fori_loop(0, n_steps, step_body, 0)

    return gather_kernel(idx_flat, table)


def _make_mlp_body(BB, L, d, aliased):
    def _mlp_body(*refs):
        x_ref, w1_ref, b1_ref, w2_ref, b2_ref, w3_ref, b3_ref = refs[:7]
        o_ref = refs[-1]
        x2 = x_ref[...]
        x = jnp.concatenate([x2[:, :d], x2[:, d:]], axis=0)
        h = jnp.dot(x, w1_ref[...], preferred_element_type=jnp.float32) + b1_ref[...]
        h = jnp.maximum(h, 0.0)
        h = jnp.dot(h, w2_ref[...], preferred_element_type=jnp.float32) + b2_ref[...]
        h = jnp.tanh(h)
        o = jnp.dot(h, w3_ref[...], preferred_element_type=jnp.float32) + b3_ref[...]
        o_t = o.T
        for l in range(L):
            o_ref[l] = o_t[:, l * BB:(l + 1) * BB]
    return _mlp_body


def _tc_mlp_chunk(embs2, w1t, b1, w2t, b2, w3t, b3, B, L, chunk, n_chunks,
                  prev_out):
    d_in = embs2.shape[1] // 2
    h1 = w1t.shape[1]
    h2 = w2t.shape[1]
    d_out = w3t.shape[1]
    BB = 4 * _C // L
    blocks = B // BB // n_chunks
    base = chunk * blocks
    in_specs = [
        pl.BlockSpec((2 * _C, 2 * d_in), lambda i: (i, 0)),
        pl.BlockSpec((d_in, h1), lambda i: (0, 0)),
        pl.BlockSpec((1, h1), lambda i: (0, 0)),
        pl.BlockSpec((h1, h2), lambda i: (0, 0)),
        pl.BlockSpec((1, h2), lambda i: (0, 0)),
        pl.BlockSpec((h2, d_out), lambda i: (0, 0)),
        pl.BlockSpec((1, d_out), lambda i: (0, 0)),
    ]
    args = [embs2, w1t, b1.reshape(1, -1), w2t, b2.reshape(1, -1), w3t,
            b3.reshape(1, -1)]
    aliases = {}
    if prev_out is not None:
        in_specs.append(pl.BlockSpec(memory_space=pl.ANY))
        args.append(prev_out)
        aliases = {7: 0}
    return pl.pallas_call(
        _make_mlp_body(BB, L, d_in, prev_out is not None),
        grid=(blocks,),
        in_specs=in_specs,
        out_specs=pl.BlockSpec((L, d_out, BB), lambda i: (0, 0, base + i)),
        out_shape=jax.ShapeDtypeStruct((L, d_out, B), jnp.float32),
        input_output_aliases=aliases,
        compiler_params=pltpu.CompilerParams(
            dimension_semantics=("arbitrary",),
        ),
    )(*args)


def kernel(indices, table, W1, b1, W2, b2, W3, b3):
    B, L = indices.shape
    d = table.shape[1]
    n_rows = B * L
    BB = 4 * _C // L
    n_chunks = 8
    rows_c = n_rows // n_chunks
    idx_perm = (
        indices.reshape(B // BB, BB, L)
        .transpose(0, 2, 1)
        .reshape(n_rows)
        .astype(jnp.int32)
    )
    w1t, w2t, w3t = W1.T, W2.T, W3.T
    stagings = [
        _sc_gather(table, lax.slice(idx_perm, (c * rows_c,), ((c + 1) * rows_c,)),
                   rows_c, d)
        for c in range(n_chunks)
    ]
    out_t = None
    for c in range(n_chunks):
        out_t = _tc_mlp_chunk(stagings[c], w1t, b1, w2t, b2, w3t, b3, B, L,
                              c, n_chunks, out_t)
    return jnp.transpose(out_t, (2, 0, 1))
